# baseline (speedup 1.0000x reference)
import functools

import jax
import jax.numpy as jnp
from jax import lax
from jax.experimental import pallas as pl
from jax.experimental.pallas import tpu as pltpu
from jax.experimental.pallas import tpu_sc as plsc

VOCAB = 1000
D_MODEL = 32
DPAD = 128
BATCH = 1024
SEQ = 50
N_TOK = BATCH * SEQ

NC, NS = 2, 16
NW = NC * NS
TOK_PER_W = N_TOK // NW
CHUNK = 80
N_CHUNK = TOK_PER_W // CHUNK
N_PAIR = N_CHUNK // 2


@functools.cache
def _make_lookup_kernel():
    mesh = plsc.VectorSubcoreMesh(core_axis_name="c", subcore_axis_name="s",
                                  num_cores=NC, num_subcores=NS)

    @functools.partial(
        pl.kernel,
        out_type=jax.ShapeDtypeStruct((N_TOK, DPAD), jnp.float32),
        mesh=mesh,
        scratch_types=[
            pltpu.VMEM((TOK_PER_W,), jnp.int32),
            pltpu.VMEM((CHUNK, DPAD), jnp.float32),
            pltpu.VMEM((CHUNK, DPAD), jnp.float32),
            pltpu.SemaphoreType.DMA,
            pltpu.SemaphoreType.DMA,
        ],
    )
    def lookup_kernel(tab_hbm, idx_hbm, h_hbm, idx_v, buf0, buf1, sem0, sem1):
        wid = lax.axis_index("s") * NC + lax.axis_index("c")
        base = wid * TOK_PER_W
        pltpu.sync_copy(idx_hbm.at[pl.ds(base, TOK_PER_W)], idx_v)

        def _start(c, buf, sem):
            pltpu.async_copy(
                tab_hbm.at[idx_v.at[pl.ds(c * CHUNK, CHUNK)]], buf, sem)

        def _finish(c, buf, sem):
            pltpu.make_async_copy(
                tab_hbm.at[idx_v.at[pl.ds(c * CHUNK, CHUNK)]],
                buf, sem).wait()
            pltpu.sync_copy(buf, h_hbm.at[pl.ds(base + c * CHUNK, CHUNK)])

        _start(0, buf0, sem0)

        def _pair(i, carry):
            c0 = 2 * i
            _start(c0 + 1, buf1, sem1)
            _finish(c0, buf0, sem0)

            @pl.when(c0 + 2 < N_CHUNK)
            def _():
                _start(c0 + 2, buf0, sem0)

            _finish(c0 + 1, buf1, sem1)
            return carry

        lax.fori_loop(0, N_PAIR, _pair, 0)

    return lookup_kernel


def _proj_body(w_ref, b_ref, h_ref, o_ref):
    o_ref[0] = lax.dot_general(
        w_ref[...], h_ref[0], (((1,), (1,)), ((), ())),
        preferred_element_type=jnp.float32) + b_ref[...]


def _project(W_pad, b, h_t):
    return pl.pallas_call(
        _proj_body,
        grid=(SEQ,),
        in_specs=[
            pl.BlockSpec((VOCAB, DPAD), lambda l: (0, 0)),
            pl.BlockSpec((VOCAB, 1), lambda l: (0, 0)),
            pl.BlockSpec((1, BATCH, DPAD), lambda l: (l, 0, 0)),
        ],
        out_specs=pl.BlockSpec((1, VOCAB, BATCH), lambda l: (l, 0, 0)),
        out_shape=jax.ShapeDtypeStruct((SEQ, VOCAB, BATCH), jnp.float32),
    )(W_pad, b.reshape(VOCAB, 1), h_t)


def kernel(x, emb, W, b):
    xf = x.T.reshape(N_TOK)
    tab = jnp.pad(emb, ((0, 0), (0, DPAD - D_MODEL)))
    w_pad = jnp.pad(W, ((0, 0), (0, DPAD - D_MODEL)))
    h = _make_lookup_kernel()(tab, xf)
    h_t = h.reshape(SEQ, BATCH, DPAD)
    out_t = _project(w_pad, b, h_t)
    return jnp.transpose(out_t, (2, 0, 1))


# speedup vs baseline: 1.1460x; 1.1460x over previous
import functools

import jax
import jax.numpy as jnp
from jax import lax
from jax.experimental import pallas as pl
from jax.experimental.pallas import tpu as pltpu
from jax.experimental.pallas import tpu_sc as plsc

VOCAB = 1000
D_MODEL = 32
BATCH = 1024
SEQ = 50
SEQ_A = 32
SEQ_B = SEQ - SEQ_A
LANES = 16
N_VEC = BATCH // LANES

NC, NS = 2, 16
NW = NC * NS


@functools.cache
def _make_lookup_kernel(l0, n_l):
    mesh = plsc.VectorSubcoreMesh(core_axis_name="c", subcore_axis_name="s",
                                  num_cores=NC, num_subcores=NS)

    @functools.partial(
        pl.kernel,
        out_type=jax.ShapeDtypeStruct((n_l, D_MODEL, BATCH), jnp.float32),
        mesh=mesh,
        compiler_params=pltpu.CompilerParams(needs_layout_passes=False),
        scratch_types=[
            pltpu.VMEM((D_MODEL, BATCH), jnp.float32),
            pltpu.VMEM((BATCH,), jnp.int32),
            pltpu.VMEM((D_MODEL, BATCH), jnp.float32),
        ],
    )
    def lookup_kernel(embt_hbm, xt_hbm, ht_hbm, tab_v, idx_v, h_v):
        wid = lax.axis_index("s") * NC + lax.axis_index("c")
        pltpu.sync_copy(embt_hbm, tab_v)

        def _one_l(l):
            pltpu.sync_copy(xt_hbm.at[l0 + l], idx_v)

            def _col_group(g, carry):
                cols = idx_v[pl.ds(g * LANES, LANES)]
                for d in range(D_MODEL):
                    rows = jnp.full((LANES,), d, dtype=jnp.int32)
                    h_v[d, pl.ds(g * LANES, LANES)] = plsc.load_gather(
                        tab_v, [rows, cols])
                return carry

            lax.fori_loop(0, N_VEC, _col_group, 0)
            pltpu.sync_copy(h_v, ht_hbm.at[l])

        if n_l == NW:
            _one_l(wid)
        else:
            @pl.when(wid < n_l)
            def _():
                _one_l(wid)

    return lookup_kernel


def _proj_init_body(w_ref, b_ref, h_ref, o_ref):
    o_ref[0] = lax.dot_general(
        w_ref[...], h_ref[0], (((1,), (0,)), ((), ())),
        preferred_element_type=jnp.float32) + b_ref[...]


def _proj_update_body(w_ref, b_ref, h_ref, _, o_ref):
    o_ref[0] = lax.dot_general(
        w_ref[...], h_ref[0], (((1,), (0,)), ((), ())),
        preferred_element_type=jnp.float32) + b_ref[...]


def _project_a(W, b2, h_a):
    return pl.pallas_call(
        _proj_init_body,
        grid=(SEQ_A,),
        in_specs=[
            pl.BlockSpec((VOCAB, D_MODEL), lambda l: (0, 0)),
            pl.BlockSpec((VOCAB, 1), lambda l: (0, 0)),
            pl.BlockSpec((1, D_MODEL, BATCH), lambda l: (l, 0, 0)),
        ],
        out_specs=pl.BlockSpec((1, VOCAB, BATCH), lambda l: (l, 0, 0)),
        out_shape=jax.ShapeDtypeStruct((SEQ, VOCAB, BATCH), jnp.float32),
    )(W, b2, h_a)


def _project_b(W, b2, h_b, out_in):
    return pl.pallas_call(
        _proj_update_body,
        grid=(SEQ_B,),
        in_specs=[
            pl.BlockSpec((VOCAB, D_MODEL), lambda l: (0, 0)),
            pl.BlockSpec((VOCAB, 1), lambda l: (0, 0)),
            pl.BlockSpec((1, D_MODEL, BATCH), lambda l: (l, 0, 0)),
            pl.BlockSpec(memory_space=pl.ANY),
        ],
        out_specs=pl.BlockSpec((1, VOCAB, BATCH), lambda l: (SEQ_A + l, 0, 0)),
        out_shape=jax.ShapeDtypeStruct((SEQ, VOCAB, BATCH), jnp.float32),
        input_output_aliases={3: 0},
    )(W, b2, h_b, out_in)


def kernel(x, emb, W, b):
    x_t = x.T
    emb_t = jnp.pad(emb.T, ((0, 0), (0, BATCH - VOCAB)))
    h_a = _make_lookup_kernel(0, SEQ_A)(emb_t, x_t)
    h_b = _make_lookup_kernel(SEQ_A, SEQ_B)(emb_t, x_t)
    b2 = b.reshape(VOCAB, 1)
    out_t = _project_a(W, b2, h_a)
    out_t = _project_b(W, b2, h_b, out_t)
    return jnp.transpose(out_t, (2, 0, 1))
